# Initial kernel scaffold; baseline (speedup 1.0000x reference)
import jax
import jax.numpy as jnp
from jax.experimental import pallas as pl


def kernel(x, edge_index, lin1_W, lin1_b, nn1_W1, nn1_b1, nn1_W2, nn1_b2, bn1_g, bn1_b, nn2_W1, nn2_b1, nn2_W2, nn2_b2, bn2_g, bn2_b, fc1_W, fc1_b, fc2_W, fc2_b):
    raise NotImplementedError("write your pallas kernel here")


# speedup vs baseline: 3.6496x; 3.6496x over previous
import functools

import jax
import jax.numpy as jnp
from jax import lax
from jax.experimental import pallas as pl
from jax.experimental.pallas import tpu as pltpu
from jax.experimental.pallas import tpu_sc as plsc

F32 = jnp.float32


def _sc_segment_sum(table_a, table_b, src, dst):
    n, w = table_a.shape
    e = src.shape[0]
    n_tiles = 16
    ch = 128
    n_chunk = e // ch
    rch = 400
    n_rchunk = n // rch
    assert ch * n_chunk == e and rch * n_rchunk == n

    mesh = plsc.VectorSubcoreMesh(core_axis_name="c", subcore_axis_name="s")

    @functools.partial(
        pl.kernel,
        mesh=mesh,
        compiler_params=pltpu.CompilerParams(use_tc_tiling_on_sc=False),
        out_type=(
            jax.ShapeDtypeStruct((n, w), F32),
            jax.ShapeDtypeStruct((n, w), F32),
        ),
        scratch_types=[
            pltpu.VMEM_SHARED((n, w), F32),
            pltpu.VMEM((ch,), jnp.int32),
            pltpu.VMEM((ch,), jnp.int32),
            pltpu.VMEM((ch, w), F32),
            pltpu.VMEM((rch, w), F32),
            pltpu.SemaphoreType.DMA,
        ],
    )
    def seg_kernel(ta, tb, src_h, dst_h, out_a, out_b,
                   acc, sidx, didx, rows, zbuf, sem):
        c = lax.axis_index("c")
        s = lax.axis_index("s")
        zvec = jnp.zeros((16,), F32)

        def zero_row(i, carry):
            for k in range(w // 16):
                zbuf[i, pl.ds(16 * k, 16)] = zvec
            return carry

        lax.fori_loop(0, rch, zero_row, 0)

        def zero_acc(j, carry):
            q = s + n_tiles * j

            @pl.when(q < n_rchunk)
            def _():
                pltpu.sync_copy(zbuf, acc.at[pl.ds(q * rch, rch)])

            return carry

        lax.fori_loop(0, (n_rchunk + n_tiles - 1) // n_tiles, zero_acc, 0)
        plsc.subcore_barrier()

        def run(table, out):
            def chunk(g, carry):
                q = s + n_tiles * g

                @pl.when(q < n_chunk)
                def _():
                    b = q * ch
                    pltpu.sync_copy(src_h.at[pl.ds(b, ch)], sidx)
                    pltpu.sync_copy(dst_h.at[pl.ds(b, ch)], didx)
                    pltpu.async_copy(table.at[sidx], rows, sem).wait()
                    pltpu.sync_copy(rows, acc.at[didx], add=True)

                return carry

            lax.fori_loop(0, (n_chunk + n_tiles - 1) // n_tiles, chunk, 0)
            plsc.subcore_barrier()

            def wb(j, carry):
                q = s + n_tiles * j

                @pl.when(q < n_rchunk)
                def _():
                    pltpu.sync_copy(acc.at[pl.ds(q * rch, rch)],
                                    out.at[pl.ds(q * rch, rch)])

                return carry

            lax.fori_loop(0, (n_rchunk + n_tiles - 1) // n_tiles, wb, 0)

        @pl.when(c == 0)
        def _():
            run(ta, out_a)

        @pl.when(c == 1)
        def _():
            run(tb, out_b)

    return seg_kernel(table_a, table_b, src, dst)


_BN_ROWS = 2000


def _lin1(x, w, b):
    n = x.shape[0]
    d_in, d_out = w.shape
    half = d_out // 2

    def body(x_ref, w_ref, b_ref, oa_ref, ob_ref):
        h = jnp.dot(x_ref[...], w_ref[...], preferred_element_type=F32)
        h = h + b_ref[...]
        oa_ref[...] = h[:, :half]
        ob_ref[...] = h[:, half:]

    return pl.pallas_call(
        body,
        grid=(n // _BN_ROWS,),
        in_specs=[
            pl.BlockSpec((_BN_ROWS, d_in), lambda i: (i, 0)),
            pl.BlockSpec((d_in, d_out), lambda i: (0, 0)),
            pl.BlockSpec((1, d_out), lambda i: (0, 0)),
        ],
        out_specs=[pl.BlockSpec((_BN_ROWS, half), lambda i: (i, 0))] * 2,
        out_shape=[jax.ShapeDtypeStruct((n, half), F32)] * 2,
    )(x, w, b.reshape(1, d_out))


def _gin_mlp(agg_a, agg_b, h_a, h_b, w1, b1, w2, b2):
    n, half = agg_a.shape
    hid = w1.shape[1]
    d_out = w2.shape[1]

    def body(aa, ab, ha, hb, w1_ref, b1_ref, w2_ref, b2_ref,
             t_ref, sum_ref, sq_ref):
        i = pl.program_id(0)
        z = jnp.concatenate([aa[...] + ha[...], ab[...] + hb[...]], axis=1)
        u = jnp.maximum(
            jnp.dot(z, w1_ref[...], preferred_element_type=F32) + b1_ref[...],
            0.0)
        t = jnp.dot(u, w2_ref[...], preferred_element_type=F32) + b2_ref[...]
        t_ref[...] = t

        @pl.when(i == 0)
        def _():
            sum_ref[...] = jnp.zeros_like(sum_ref)
            sq_ref[...] = jnp.zeros_like(sq_ref)

        sum_ref[...] += jnp.sum(t, axis=0, keepdims=True)
        sq_ref[...] += jnp.sum(t * t, axis=0, keepdims=True)

    return pl.pallas_call(
        body,
        grid=(n // _BN_ROWS,),
        in_specs=[
            pl.BlockSpec((_BN_ROWS, half), lambda i: (i, 0)),
            pl.BlockSpec((_BN_ROWS, half), lambda i: (i, 0)),
            pl.BlockSpec((_BN_ROWS, half), lambda i: (i, 0)),
            pl.BlockSpec((_BN_ROWS, half), lambda i: (i, 0)),
            pl.BlockSpec((2 * half, hid), lambda i: (0, 0)),
            pl.BlockSpec((1, hid), lambda i: (0, 0)),
            pl.BlockSpec((hid, d_out), lambda i: (0, 0)),
            pl.BlockSpec((1, d_out), lambda i: (0, 0)),
        ],
        out_specs=[
            pl.BlockSpec((_BN_ROWS, d_out), lambda i: (i, 0)),
            pl.BlockSpec((1, d_out), lambda i: (0, 0)),
            pl.BlockSpec((1, d_out), lambda i: (0, 0)),
        ],
        out_shape=[
            jax.ShapeDtypeStruct((n, d_out), F32),
            jax.ShapeDtypeStruct((1, d_out), F32),
            jax.ShapeDtypeStruct((1, d_out), F32),
        ],
    )(agg_a, agg_b, h_a, h_b, w1, b1.reshape(1, hid), w2, b2.reshape(1, d_out))


def _bn_apply_split(t, t_sum, t_sq, g, b):
    n, d = t.shape
    half = d // 2

    def body(t_ref, sum_ref, sq_ref, g_ref, b_ref, oa_ref, ob_ref):
        m = sum_ref[...] / n
        v = sq_ref[...] / n - m * m
        h = (t_ref[...] - m) * lax.rsqrt(v + 1e-5) * g_ref[...] + b_ref[...]
        oa_ref[...] = h[:, :half]
        ob_ref[...] = h[:, half:]

    return pl.pallas_call(
        body,
        grid=(n // _BN_ROWS,),
        in_specs=[
            pl.BlockSpec((_BN_ROWS, d), lambda i: (i, 0)),
            pl.BlockSpec((1, d), lambda i: (0, 0)),
            pl.BlockSpec((1, d), lambda i: (0, 0)),
            pl.BlockSpec((1, d), lambda i: (0, 0)),
            pl.BlockSpec((1, d), lambda i: (0, 0)),
        ],
        out_specs=[pl.BlockSpec((_BN_ROWS, half), lambda i: (i, 0))] * 2,
        out_shape=[jax.ShapeDtypeStruct((n, half), F32)] * 2,
    )(t, t_sum, t_sq, g.reshape(1, d), b.reshape(1, d))


def _bn_head(t, t_sum, t_sq, g, b, fc1_w, fc1_b, fc2_w, fc2_b):
    n, d = t.shape
    hid = fc1_w.shape[1]
    n_cls = fc2_w.shape[1]

    def body(t_ref, sum_ref, sq_ref, g_ref, b_ref,
             w1_ref, b1_ref, w2_ref, b2_ref, o_ref):
        m = sum_ref[...] / n
        v = sq_ref[...] / n - m * m
        h2 = (t_ref[...] - m) * lax.rsqrt(v + 1e-5) * g_ref[...] + b_ref[...]
        h3 = jnp.maximum(
            jnp.dot(h2, w1_ref[...], preferred_element_type=F32) + b1_ref[...],
            0.0)
        o_ref[...] = (jnp.dot(h3, w2_ref[...], preferred_element_type=F32)
                      + b2_ref[...])

    return pl.pallas_call(
        body,
        grid=(n // _BN_ROWS,),
        in_specs=[
            pl.BlockSpec((_BN_ROWS, d), lambda i: (i, 0)),
            pl.BlockSpec((1, d), lambda i: (0, 0)),
            pl.BlockSpec((1, d), lambda i: (0, 0)),
            pl.BlockSpec((1, d), lambda i: (0, 0)),
            pl.BlockSpec((1, d), lambda i: (0, 0)),
            pl.BlockSpec((d, hid), lambda i: (0, 0)),
            pl.BlockSpec((1, hid), lambda i: (0, 0)),
            pl.BlockSpec((hid, n_cls), lambda i: (0, 0)),
            pl.BlockSpec((1, n_cls), lambda i: (0, 0)),
        ],
        out_specs=pl.BlockSpec((_BN_ROWS, n_cls), lambda i: (i, 0)),
        out_shape=jax.ShapeDtypeStruct((n, n_cls), F32),
    )(t, t_sum, t_sq, g.reshape(1, d), b.reshape(1, d),
      fc1_w, fc1_b.reshape(1, hid), fc2_w, fc2_b.reshape(1, n_cls))


def kernel(x, edge_index, lin1_W, lin1_b, nn1_W1, nn1_b1, nn1_W2, nn1_b2,
           bn1_g, bn1_b, nn2_W1, nn2_b1, nn2_W2, nn2_b2, bn2_g, bn2_b,
           fc1_W, fc1_b, fc2_W, fc2_b):
    src = edge_index[0]
    dst = edge_index[1]

    h_a, h_b = _lin1(x, lin1_W, lin1_b)
    agg_a, agg_b = _sc_segment_sum(h_a, h_b, src, dst)
    t1, s1, q1 = _gin_mlp(agg_a, agg_b, h_a, h_b, nn1_W1, nn1_b1, nn1_W2, nn1_b2)
    h1_a, h1_b = _bn_apply_split(t1, s1, q1, bn1_g, bn1_b)
    a2_a, a2_b = _sc_segment_sum(h1_a, h1_b, src, dst)
    t2, s2, q2 = _gin_mlp(a2_a, a2_b, h1_a, h1_b, nn2_W1, nn2_b1, nn2_W2, nn2_b2)
    return _bn_head(t2, s2, q2, bn2_g, bn2_b, fc1_W, fc1_b, fc2_W, fc2_b)


# speedup vs baseline: 8.3891x; 2.2987x over previous
import functools

import jax
import jax.numpy as jnp
from jax import lax
from jax.experimental import pallas as pl
from jax.experimental.pallas import tpu as pltpu
from jax.experimental.pallas import tpu_sc as plsc

F32 = jnp.float32
_G = 4


def _sc_segment_sum(table_a, table_b, src, dst):
    n, w = table_a.shape
    e = src.shape[0]
    n_tiles = 16
    ch = 128
    n_chunk = e // ch
    rch = 200
    n_rchunk = n // rch
    assert ch * n_chunk == e and rch * n_rchunk == n

    mesh = plsc.VectorSubcoreMesh(core_axis_name="c", subcore_axis_name="s")

    @functools.partial(
        pl.kernel,
        mesh=mesh,
        compiler_params=pltpu.CompilerParams(use_tc_tiling_on_sc=False),
        out_type=(
            jax.ShapeDtypeStruct((n, w), F32),
            jax.ShapeDtypeStruct((n, w), F32),
        ),
        scratch_types=[
            pltpu.VMEM_SHARED((n, w), F32),
            [pltpu.VMEM((ch,), jnp.int32)] * _G,
            [pltpu.VMEM((ch,), jnp.int32)] * _G,
            [pltpu.VMEM((ch, w), F32)] * _G,
            pltpu.VMEM((rch, w), F32),
            pltpu.SemaphoreType.DMA,
            pltpu.SemaphoreType.DMA,
            pltpu.SemaphoreType.DMA,
        ],
    )
    def seg_kernel(ta, tb, src_h, dst_h, out_a, out_b,
                   acc, sidx, didx, rows, zbuf, isem, gsem, ssem):
        c = lax.axis_index("c")
        s = lax.axis_index("s")
        zvec = jnp.zeros((16,), F32)

        def zero_row(i, carry):
            for k in range(w // 16):
                zbuf[i, pl.ds(16 * k, 16)] = zvec
            return carry

        lax.fori_loop(0, rch, zero_row, 0)

        def zero_acc(j, carry):
            q = s + n_tiles * j

            @pl.when(q < n_rchunk)
            def _():
                pltpu.sync_copy(zbuf, acc.at[pl.ds(q * rch, rch)])

            return carry

        lax.fori_loop(0, (n_rchunk + n_tiles - 1) // n_tiles, zero_acc, 0)
        plsc.subcore_barrier()

        def run(table, out):
            per_tile = (n_chunk + n_tiles - 1) // n_tiles
            n_grp = (per_tile + _G - 1) // _G

            def grp(g, carry):
                qs = [s + n_tiles * (g * _G + j) for j in range(_G)]
                for j, q in enumerate(qs):
                    @pl.when(q < n_chunk)
                    def _(q=q, j=j):
                        b = q * ch
                        pltpu.async_copy(src_h.at[pl.ds(b, ch)], sidx[j], isem)
                        pltpu.async_copy(dst_h.at[pl.ds(b, ch)], didx[j], isem)
                for j, q in enumerate(qs):
                    @pl.when(q < n_chunk)
                    def _(q=q, j=j):
                        b = q * ch
                        pltpu.make_async_copy(
                            src_h.at[pl.ds(b, ch)], sidx[j], isem).wait()
                        pltpu.make_async_copy(
                            dst_h.at[pl.ds(b, ch)], didx[j], isem).wait()
                        pltpu.async_copy(table.at[sidx[j]], rows[j], gsem)
                for j, q in enumerate(qs):
                    @pl.when(q < n_chunk)
                    def _(q=q, j=j):
                        pltpu.make_async_copy(
                            table.at[sidx[j]], rows[j], gsem).wait()
                        pltpu.async_copy(rows[j], acc.at[didx[j]], ssem,
                                         add=True)
                for j, q in enumerate(qs):
                    @pl.when(q < n_chunk)
                    def _(q=q, j=j):
                        pltpu.make_async_copy(rows[j], acc.at[didx[j]],
                                              ssem).wait()
                return carry

            lax.fori_loop(0, n_grp, grp, 0)
            plsc.subcore_barrier()

            def wb(j, carry):
                q = s + n_tiles * j

                @pl.when(q < n_rchunk)
                def _():
                    pltpu.sync_copy(acc.at[pl.ds(q * rch, rch)],
                                    out.at[pl.ds(q * rch, rch)])

                return carry

            lax.fori_loop(0, (n_rchunk + n_tiles - 1) // n_tiles, wb, 0)

        @pl.when(c == 0)
        def _():
            run(ta, out_a)

        @pl.when(c == 1)
        def _():
            run(tb, out_b)

    return seg_kernel(table_a, table_b, src, dst)


_BN_ROWS = 2000


def _lin1(x, w, b):
    n = x.shape[0]
    d_in, d_out = w.shape
    half = d_out // 2

    def body(x_ref, w_ref, b_ref, oa_ref, ob_ref):
        h = jnp.dot(x_ref[...], w_ref[...], preferred_element_type=F32)
        h = h + b_ref[...]
        oa_ref[...] = h[:, :half]
        ob_ref[...] = h[:, half:]

    return pl.pallas_call(
        body,
        grid=(n // _BN_ROWS,),
        in_specs=[
            pl.BlockSpec((_BN_ROWS, d_in), lambda i: (i, 0)),
            pl.BlockSpec((d_in, d_out), lambda i: (0, 0)),
            pl.BlockSpec((1, d_out), lambda i: (0, 0)),
        ],
        out_specs=[pl.BlockSpec((_BN_ROWS, half), lambda i: (i, 0))] * 2,
        out_shape=[jax.ShapeDtypeStruct((n, half), F32)] * 2,
    )(x, w, b.reshape(1, d_out))


def _gin_mlp(agg_a, agg_b, h_a, h_b, w1, b1, w2, b2):
    n, half = agg_a.shape
    hid = w1.shape[1]
    d_out = w2.shape[1]

    def body(aa, ab, ha, hb, w1_ref, b1_ref, w2_ref, b2_ref,
             t_ref, sum_ref, sq_ref):
        i = pl.program_id(0)
        z = jnp.concatenate([aa[...] + ha[...], ab[...] + hb[...]], axis=1)
        u = jnp.maximum(
            jnp.dot(z, w1_ref[...], preferred_element_type=F32) + b1_ref[...],
            0.0)
        t = jnp.dot(u, w2_ref[...], preferred_element_type=F32) + b2_ref[...]
        t_ref[...] = t

        @pl.when(i == 0)
        def _():
            sum_ref[...] = jnp.zeros_like(sum_ref)
            sq_ref[...] = jnp.zeros_like(sq_ref)

        sum_ref[...] += jnp.sum(t, axis=0, keepdims=True)
        sq_ref[...] += jnp.sum(t * t, axis=0, keepdims=True)

    return pl.pallas_call(
        body,
        grid=(n // _BN_ROWS,),
        in_specs=[
            pl.BlockSpec((_BN_ROWS, half), lambda i: (i, 0)),
            pl.BlockSpec((_BN_ROWS, half), lambda i: (i, 0)),
            pl.BlockSpec((_BN_ROWS, half), lambda i: (i, 0)),
            pl.BlockSpec((_BN_ROWS, half), lambda i: (i, 0)),
            pl.BlockSpec((2 * half, hid), lambda i: (0, 0)),
            pl.BlockSpec((1, hid), lambda i: (0, 0)),
            pl.BlockSpec((hid, d_out), lambda i: (0, 0)),
            pl.BlockSpec((1, d_out), lambda i: (0, 0)),
        ],
        out_specs=[
            pl.BlockSpec((_BN_ROWS, d_out), lambda i: (i, 0)),
            pl.BlockSpec((1, d_out), lambda i: (0, 0)),
            pl.BlockSpec((1, d_out), lambda i: (0, 0)),
        ],
        out_shape=[
            jax.ShapeDtypeStruct((n, d_out), F32),
            jax.ShapeDtypeStruct((1, d_out), F32),
            jax.ShapeDtypeStruct((1, d_out), F32),
        ],
    )(agg_a, agg_b, h_a, h_b, w1, b1.reshape(1, hid), w2, b2.reshape(1, d_out))


def _bn_apply_split(t, t_sum, t_sq, g, b):
    n, d = t.shape
    half = d // 2

    def body(t_ref, sum_ref, sq_ref, g_ref, b_ref, oa_ref, ob_ref):
        m = sum_ref[...] / n
        v = sq_ref[...] / n - m * m
        h = (t_ref[...] - m) * lax.rsqrt(v + 1e-5) * g_ref[...] + b_ref[...]
        oa_ref[...] = h[:, :half]
        ob_ref[...] = h[:, half:]

    return pl.pallas_call(
        body,
        grid=(n // _BN_ROWS,),
        in_specs=[
            pl.BlockSpec((_BN_ROWS, d), lambda i: (i, 0)),
            pl.BlockSpec((1, d), lambda i: (0, 0)),
            pl.BlockSpec((1, d), lambda i: (0, 0)),
            pl.BlockSpec((1, d), lambda i: (0, 0)),
            pl.BlockSpec((1, d), lambda i: (0, 0)),
        ],
        out_specs=[pl.BlockSpec((_BN_ROWS, half), lambda i: (i, 0))] * 2,
        out_shape=[jax.ShapeDtypeStruct((n, half), F32)] * 2,
    )(t, t_sum, t_sq, g.reshape(1, d), b.reshape(1, d))


def _bn_head(t, t_sum, t_sq, g, b, fc1_w, fc1_b, fc2_w, fc2_b):
    n, d = t.shape
    hid = fc1_w.shape[1]
    n_cls = fc2_w.shape[1]

    def body(t_ref, sum_ref, sq_ref, g_ref, b_ref,
             w1_ref, b1_ref, w2_ref, b2_ref, o_ref):
        m = sum_ref[...] / n
        v = sq_ref[...] / n - m * m
        h2 = (t_ref[...] - m) * lax.rsqrt(v + 1e-5) * g_ref[...] + b_ref[...]
        h3 = jnp.maximum(
            jnp.dot(h2, w1_ref[...], preferred_element_type=F32) + b1_ref[...],
            0.0)
        o_ref[...] = (jnp.dot(h3, w2_ref[...], preferred_element_type=F32)
                      + b2_ref[...])

    return pl.pallas_call(
        body,
        grid=(n // _BN_ROWS,),
        in_specs=[
            pl.BlockSpec((_BN_ROWS, d), lambda i: (i, 0)),
            pl.BlockSpec((1, d), lambda i: (0, 0)),
            pl.BlockSpec((1, d), lambda i: (0, 0)),
            pl.BlockSpec((1, d), lambda i: (0, 0)),
            pl.BlockSpec((1, d), lambda i: (0, 0)),
            pl.BlockSpec((d, hid), lambda i: (0, 0)),
            pl.BlockSpec((1, hid), lambda i: (0, 0)),
            pl.BlockSpec((hid, n_cls), lambda i: (0, 0)),
            pl.BlockSpec((1, n_cls), lambda i: (0, 0)),
        ],
        out_specs=pl.BlockSpec((_BN_ROWS, n_cls), lambda i: (i, 0)),
        out_shape=jax.ShapeDtypeStruct((n, n_cls), F32),
    )(t, t_sum, t_sq, g.reshape(1, d), b.reshape(1, d),
      fc1_w, fc1_b.reshape(1, hid), fc2_w, fc2_b.reshape(1, n_cls))


def kernel(x, edge_index, lin1_W, lin1_b, nn1_W1, nn1_b1, nn1_W2, nn1_b2,
           bn1_g, bn1_b, nn2_W1, nn2_b1, nn2_W2, nn2_b2, bn2_g, bn2_b,
           fc1_W, fc1_b, fc2_W, fc2_b):
    src = edge_index[0]
    dst = edge_index[1]

    h_a, h_b = _lin1(x, lin1_W, lin1_b)
    agg_a, agg_b = _sc_segment_sum(h_a, h_b, src, dst)
    t1, s1, q1 = _gin_mlp(agg_a, agg_b, h_a, h_b, nn1_W1, nn1_b1, nn1_W2, nn1_b2)
    h1_a, h1_b = _bn_apply_split(t1, s1, q1, bn1_g, bn1_b)
    a2_a, a2_b = _sc_segment_sum(h1_a, h1_b, src, dst)
    t2, s2, q2 = _gin_mlp(a2_a, a2_b, h1_a, h1_b, nn2_W1, nn2_b1, nn2_W2, nn2_b2)
    return _bn_head(t2, s2, q2, bn2_g, bn2_b, fc1_W, fc1_b, fc2_W, fc2_b)


# speedup vs baseline: 8.4424x; 1.0064x over previous
import functools

import jax
import jax.numpy as jnp
from jax import lax
from jax.experimental import pallas as pl
from jax.experimental.pallas import tpu as pltpu
from jax.experimental.pallas import tpu_sc as plsc

F32 = jnp.float32
_G = 4


def _sc_segment_sum(table_a, table_b, src, dst):
    n, w = table_a.shape
    e = src.shape[0]
    n_tiles = 16
    ch = 80
    n_chunk = e // ch
    per_tile = n_chunk // n_tiles
    rch = 80
    n_rchunk = n // rch
    assert ch * n_chunk == e and per_tile * n_tiles == n_chunk
    assert rch * n_rchunk == n

    mesh = plsc.VectorSubcoreMesh(core_axis_name="c", subcore_axis_name="s")

    @functools.partial(
        pl.kernel,
        mesh=mesh,
        compiler_params=pltpu.CompilerParams(use_tc_tiling_on_sc=False),
        out_type=(
            jax.ShapeDtypeStruct((n, w), F32),
            jax.ShapeDtypeStruct((n, w), F32),
        ),
        scratch_types=[
            pltpu.VMEM_SHARED((n, w), F32),
            [pltpu.VMEM((ch,), jnp.int32)] * (2 * _G),
            [pltpu.VMEM((ch,), jnp.int32)] * (2 * _G),
            [pltpu.VMEM((ch, w), F32)] * (2 * _G),
            [pltpu.SemaphoreType.DMA] * 2,
            [pltpu.SemaphoreType.DMA] * 2,
            [pltpu.SemaphoreType.DMA] * 2,
            pltpu.SemaphoreType.DMA,
        ],
    )
    def seg_kernel(ta, tb, src_h, dst_h, out_a, out_b,
                   acc, sidx, didx, rows, isems, gsems, ssems, wsem):
        c = lax.axis_index("c")
        s = lax.axis_index("s")
        zvec = jnp.zeros((16,), F32)

        def zero_row(i, carry):
            for k in range(w // 16):
                rows[0][i, pl.ds(16 * k, 16)] = zvec
            return carry

        lax.fori_loop(0, ch, zero_row, 0)
        n_rj = (n_rchunk + n_tiles - 1) // n_tiles

        def zero_acc(j, carry):
            q = s + n_tiles * j

            @pl.when(q < n_rchunk)
            def _():
                pltpu.async_copy(rows[0], acc.at[pl.ds(q * rch, rch)], wsem)

            return carry

        def zero_drain(j, carry):
            q = s + n_tiles * j

            @pl.when(q < n_rchunk)
            def _():
                pltpu.make_async_copy(
                    rows[0], acc.at[pl.ds(q * rch, rch)], wsem).wait()

            return carry

        lax.fori_loop(0, n_rj, zero_acc, 0)
        lax.fori_loop(0, n_rj, zero_drain, 0)
        plsc.subcore_barrier()

        def run(table, out):
            def issue_idx(ct0, off, sem):
                for j in range(_G):
                    ct = ct0 + j

                    @pl.when(ct < per_tile)
                    def _(ct=ct, j=j):
                        b = (s + n_tiles * ct) * ch
                        pltpu.async_copy(src_h.at[pl.ds(b, ch)],
                                         sidx[off + j], sem)
                        pltpu.async_copy(dst_h.at[pl.ds(b, ch)],
                                         didx[off + j], sem)

            def gathers(ct0, off, isem, gsem):
                for j in range(_G):
                    ct = ct0 + j

                    @pl.when(ct < per_tile)
                    def _(ct=ct, j=j):
                        b = (s + n_tiles * ct) * ch
                        pltpu.make_async_copy(src_h.at[pl.ds(b, ch)],
                                              sidx[off + j], isem).wait()
                        pltpu.make_async_copy(dst_h.at[pl.ds(b, ch)],
                                              didx[off + j], isem).wait()
                        pltpu.async_copy(table.at[sidx[off + j]],
                                         rows[off + j], gsem)

            def scatters(ct0, off, gsem, ssem):
                for j in range(_G):
                    ct = ct0 + j

                    @pl.when(ct < per_tile)
                    def _(ct=ct, j=j):
                        pltpu.make_async_copy(table.at[sidx[off + j]],
                                              rows[off + j], gsem).wait()
                        pltpu.async_copy(rows[off + j],
                                         acc.at[didx[off + j]], ssem,
                                         add=True)

            def drain_scatters(ct0, off, ssem):
                for j in range(_G):
                    ct = ct0 + j

                    @pl.when(ct < per_tile)
                    def _(ct=ct, j=j):
                        pltpu.make_async_copy(rows[off + j],
                                              acc.at[didx[off + j]],
                                              ssem).wait()

            def two_grp(t, carry):
                ca = 2 * _G * t
                cb = ca + _G
                issue_idx(ca, 0, isems[0])
                issue_idx(cb, _G, isems[1])
                gathers(ca, 0, isems[0], gsems[0])
                scatters(ca, 0, gsems[0], ssems[0])
                gathers(cb, _G, isems[1], gsems[1])
                drain_scatters(ca, 0, ssems[0])
                scatters(cb, _G, gsems[1], ssems[1])
                drain_scatters(cb, _G, ssems[1])
                return carry

            lax.fori_loop(0, (per_tile + 2 * _G - 1) // (2 * _G), two_grp, 0)
            plsc.subcore_barrier()

            def wb(j, carry):
                q = s + n_tiles * j

                @pl.when(q < n_rchunk)
                def _():
                    pltpu.async_copy(acc.at[pl.ds(q * rch, rch)],
                                     out.at[pl.ds(q * rch, rch)], wsem)

                return carry

            def wb_drain(j, carry):
                q = s + n_tiles * j

                @pl.when(q < n_rchunk)
                def _():
                    pltpu.make_async_copy(acc.at[pl.ds(q * rch, rch)],
                                          out.at[pl.ds(q * rch, rch)],
                                          wsem).wait()

                return carry

            lax.fori_loop(0, n_rj, wb, 0)
            lax.fori_loop(0, n_rj, wb_drain, 0)

        @pl.when(c == 0)
        def _():
            run(ta, out_a)

        @pl.when(c == 1)
        def _():
            run(tb, out_b)

    return seg_kernel(table_a, table_b, src, dst)


_BN_ROWS = 2000


def _lin1(x, w, b):
    n = x.shape[0]
    d_in, d_out = w.shape
    half = d_out // 2

    def body(x_ref, w_ref, b_ref, oa_ref, ob_ref):
        h = jnp.dot(x_ref[...], w_ref[...], preferred_element_type=F32)
        h = h + b_ref[...]
        oa_ref[...] = h[:, :half]
        ob_ref[...] = h[:, half:]

    return pl.pallas_call(
        body,
        grid=(n // _BN_ROWS,),
        in_specs=[
            pl.BlockSpec((_BN_ROWS, d_in), lambda i: (i, 0)),
            pl.BlockSpec((d_in, d_out), lambda i: (0, 0)),
            pl.BlockSpec((1, d_out), lambda i: (0, 0)),
        ],
        out_specs=[pl.BlockSpec((_BN_ROWS, half), lambda i: (i, 0))] * 2,
        out_shape=[jax.ShapeDtypeStruct((n, half), F32)] * 2,
    )(x, w, b.reshape(1, d_out))


def _gin_mlp(agg_a, agg_b, h_a, h_b, w1, b1, w2, b2):
    n, half = agg_a.shape
    hid = w1.shape[1]
    d_out = w2.shape[1]

    def body(aa, ab, ha, hb, w1_ref, b1_ref, w2_ref, b2_ref,
             t_ref, sum_ref, sq_ref):
        i = pl.program_id(0)
        z = jnp.concatenate([aa[...] + ha[...], ab[...] + hb[...]], axis=1)
        u = jnp.maximum(
            jnp.dot(z, w1_ref[...], preferred_element_type=F32) + b1_ref[...],
            0.0)
        t = jnp.dot(u, w2_ref[...], preferred_element_type=F32) + b2_ref[...]
        t_ref[...] = t

        @pl.when(i == 0)
        def _():
            sum_ref[...] = jnp.zeros_like(sum_ref)
            sq_ref[...] = jnp.zeros_like(sq_ref)

        sum_ref[...] += jnp.sum(t, axis=0, keepdims=True)
        sq_ref[...] += jnp.sum(t * t, axis=0, keepdims=True)

    return pl.pallas_call(
        body,
        grid=(n // _BN_ROWS,),
        in_specs=[
            pl.BlockSpec((_BN_ROWS, half), lambda i: (i, 0)),
            pl.BlockSpec((_BN_ROWS, half), lambda i: (i, 0)),
            pl.BlockSpec((_BN_ROWS, half), lambda i: (i, 0)),
            pl.BlockSpec((_BN_ROWS, half), lambda i: (i, 0)),
            pl.BlockSpec((2 * half, hid), lambda i: (0, 0)),
            pl.BlockSpec((1, hid), lambda i: (0, 0)),
            pl.BlockSpec((hid, d_out), lambda i: (0, 0)),
            pl.BlockSpec((1, d_out), lambda i: (0, 0)),
        ],
        out_specs=[
            pl.BlockSpec((_BN_ROWS, d_out), lambda i: (i, 0)),
            pl.BlockSpec((1, d_out), lambda i: (0, 0)),
            pl.BlockSpec((1, d_out), lambda i: (0, 0)),
        ],
        out_shape=[
            jax.ShapeDtypeStruct((n, d_out), F32),
            jax.ShapeDtypeStruct((1, d_out), F32),
            jax.ShapeDtypeStruct((1, d_out), F32),
        ],
    )(agg_a, agg_b, h_a, h_b, w1, b1.reshape(1, hid), w2, b2.reshape(1, d_out))


def _bn_apply_split(t, t_sum, t_sq, g, b):
    n, d = t.shape
    half = d // 2

    def body(t_ref, sum_ref, sq_ref, g_ref, b_ref, oa_ref, ob_ref):
        m = sum_ref[...] / n
        v = sq_ref[...] / n - m * m
        h = (t_ref[...] - m) * lax.rsqrt(v + 1e-5) * g_ref[...] + b_ref[...]
        oa_ref[...] = h[:, :half]
        ob_ref[...] = h[:, half:]

    return pl.pallas_call(
        body,
        grid=(n // _BN_ROWS,),
        in_specs=[
            pl.BlockSpec((_BN_ROWS, d), lambda i: (i, 0)),
            pl.BlockSpec((1, d), lambda i: (0, 0)),
            pl.BlockSpec((1, d), lambda i: (0, 0)),
            pl.BlockSpec((1, d), lambda i: (0, 0)),
            pl.BlockSpec((1, d), lambda i: (0, 0)),
        ],
        out_specs=[pl.BlockSpec((_BN_ROWS, half), lambda i: (i, 0))] * 2,
        out_shape=[jax.ShapeDtypeStruct((n, half), F32)] * 2,
    )(t, t_sum, t_sq, g.reshape(1, d), b.reshape(1, d))


def _bn_head(t, t_sum, t_sq, g, b, fc1_w, fc1_b, fc2_w, fc2_b):
    n, d = t.shape
    hid = fc1_w.shape[1]
    n_cls = fc2_w.shape[1]

    def body(t_ref, sum_ref, sq_ref, g_ref, b_ref,
             w1_ref, b1_ref, w2_ref, b2_ref, o_ref):
        m = sum_ref[...] / n
        v = sq_ref[...] / n - m * m
        h2 = (t_ref[...] - m) * lax.rsqrt(v + 1e-5) * g_ref[...] + b_ref[...]
        h3 = jnp.maximum(
            jnp.dot(h2, w1_ref[...], preferred_element_type=F32) + b1_ref[...],
            0.0)
        o = (jnp.dot(h3, w2_ref[...], preferred_element_type=F32)
             + b2_ref[...])
        o_ref[...] = o.T

    out_t = pl.pallas_call(
        body,
        out_shape=jax.ShapeDtypeStruct((n_cls, n), F32),
    )(t, t_sum, t_sq, g.reshape(1, d), b.reshape(1, d),
      fc1_w, fc1_b.reshape(1, hid), fc2_w, fc2_b.reshape(1, n_cls))
    return out_t.T


def kernel(x, edge_index, lin1_W, lin1_b, nn1_W1, nn1_b1, nn1_W2, nn1_b2,
           bn1_g, bn1_b, nn2_W1, nn2_b1, nn2_W2, nn2_b2, bn2_g, bn2_b,
           fc1_W, fc1_b, fc2_W, fc2_b):
    src = edge_index[0]
    dst = edge_index[1]

    h_a, h_b = _lin1(x, lin1_W, lin1_b)
    agg_a, agg_b = _sc_segment_sum(h_a, h_b, src, dst)
    t1, s1, q1 = _gin_mlp(agg_a, agg_b, h_a, h_b, nn1_W1, nn1_b1, nn1_W2, nn1_b2)
    h1_a, h1_b = _bn_apply_split(t1, s1, q1, bn1_g, bn1_b)
    a2_a, a2_b = _sc_segment_sum(h1_a, h1_b, src, dst)
    t2, s2, q2 = _gin_mlp(a2_a, a2_b, h1_a, h1_b, nn2_W1, nn2_b1, nn2_W2, nn2_b2)
    return _bn_head(t2, s2, q2, bn2_g, bn2_b, fc1_W, fc1_b, fc2_W, fc2_b)


# speedup vs baseline: 8.8870x; 1.0527x over previous
import functools

import jax
import jax.numpy as jnp
from jax import lax
from jax.experimental import pallas as pl
from jax.experimental.pallas import tpu as pltpu
from jax.experimental.pallas import tpu_sc as plsc

F32 = jnp.float32
_G = 3


def _sc_segment_sum(table_a, table_b, src, dst):
    n, w = table_a.shape
    e = src.shape[0]
    n_tiles = 16
    ch = 128
    n_chunk = e // ch
    per_tile = (n_chunk + n_tiles - 1) // n_tiles
    rch = 80
    n_rchunk = n // rch
    assert ch * n_chunk == e
    assert rch * n_rchunk == n

    mesh = plsc.VectorSubcoreMesh(core_axis_name="c", subcore_axis_name="s")

    @functools.partial(
        pl.kernel,
        mesh=mesh,
        compiler_params=pltpu.CompilerParams(use_tc_tiling_on_sc=False),
        out_type=(
            jax.ShapeDtypeStruct((n, w), F32),
            jax.ShapeDtypeStruct((n, w), F32),
        ),
        scratch_types=[
            pltpu.VMEM_SHARED((n, w), F32),
            [pltpu.VMEM((ch,), jnp.int32)] * (2 * _G),
            [pltpu.VMEM((ch,), jnp.int32)] * (2 * _G),
            [pltpu.VMEM((ch, w), F32)] * (2 * _G),
            [pltpu.SemaphoreType.DMA] * 2,
            [pltpu.SemaphoreType.DMA] * 2,
            [pltpu.SemaphoreType.DMA] * 2,
            pltpu.SemaphoreType.DMA,
        ],
    )
    def seg_kernel(ta, tb, src_h, dst_h, out_a, out_b,
                   acc, sidx, didx, rows, isems, gsems, ssems, wsem):
        c = lax.axis_index("c")
        s = lax.axis_index("s")
        zvec = jnp.zeros((16,), F32)

        def zero_row(i, carry):
            for k in range(w // 16):
                rows[0][i, pl.ds(16 * k, 16)] = zvec
            return carry

        lax.fori_loop(0, rch, zero_row, 0)
        n_rj = (n_rchunk + n_tiles - 1) // n_tiles

        def zero_acc(j, carry):
            q = s + n_tiles * j

            @pl.when(q < n_rchunk)
            def _():
                pltpu.async_copy(rows[0].at[pl.ds(0, rch)],
                                 acc.at[pl.ds(q * rch, rch)], wsem)

            return carry

        def zero_drain(j, carry):
            q = s + n_tiles * j

            @pl.when(q < n_rchunk)
            def _():
                pltpu.make_async_copy(
                    rows[0].at[pl.ds(0, rch)],
                    acc.at[pl.ds(q * rch, rch)], wsem).wait()

            return carry

        lax.fori_loop(0, n_rj, zero_acc, 0)
        lax.fori_loop(0, n_rj, zero_drain, 0)
        plsc.subcore_barrier()

        def run(table, out):
            def issue_idx(ct0, off, sem):
                for j in range(_G):
                    ct = ct0 + j

                    @pl.when(s + n_tiles * ct < n_chunk)
                    def _(ct=ct, j=j):
                        b = (s + n_tiles * ct) * ch
                        pltpu.async_copy(src_h.at[pl.ds(b, ch)],
                                         sidx[off + j], sem)
                        pltpu.async_copy(dst_h.at[pl.ds(b, ch)],
                                         didx[off + j], sem)

            def gathers(ct0, off, isem, gsem):
                for j in range(_G):
                    ct = ct0 + j

                    @pl.when(s + n_tiles * ct < n_chunk)
                    def _(ct=ct, j=j):
                        b = (s + n_tiles * ct) * ch
                        pltpu.make_async_copy(src_h.at[pl.ds(b, ch)],
                                              sidx[off + j], isem).wait()
                        pltpu.make_async_copy(dst_h.at[pl.ds(b, ch)],
                                              didx[off + j], isem).wait()
                        pltpu.async_copy(table.at[sidx[off + j]],
                                         rows[off + j], gsem)

            def scatters(ct0, off, gsem, ssem):
                for j in range(_G):
                    ct = ct0 + j

                    @pl.when(s + n_tiles * ct < n_chunk)
                    def _(ct=ct, j=j):
                        pltpu.make_async_copy(table.at[sidx[off + j]],
                                              rows[off + j], gsem).wait()
                        pltpu.async_copy(rows[off + j],
                                         acc.at[didx[off + j]], ssem,
                                         add=True)

            def drain_scatters(ct0, off, ssem):
                for j in range(_G):
                    ct = ct0 + j

                    @pl.when(s + n_tiles * ct < n_chunk)
                    def _(ct=ct, j=j):
                        pltpu.make_async_copy(rows[off + j],
                                              acc.at[didx[off + j]],
                                              ssem).wait()

            def two_grp(t, carry):
                ca = 2 * _G * t
                cb = ca + _G
                issue_idx(ca, 0, isems[0])
                issue_idx(cb, _G, isems[1])
                gathers(ca, 0, isems[0], gsems[0])
                scatters(ca, 0, gsems[0], ssems[0])
                gathers(cb, _G, isems[1], gsems[1])
                drain_scatters(ca, 0, ssems[0])
                scatters(cb, _G, gsems[1], ssems[1])
                drain_scatters(cb, _G, ssems[1])
                return carry

            lax.fori_loop(0, (per_tile + 2 * _G - 1) // (2 * _G), two_grp, 0)
            plsc.subcore_barrier()

            def wb(j, carry):
                q = s + n_tiles * j

                @pl.when(q < n_rchunk)
                def _():
                    pltpu.async_copy(acc.at[pl.ds(q * rch, rch)],
                                     out.at[pl.ds(q * rch, rch)], wsem)

                return carry

            def wb_drain(j, carry):
                q = s + n_tiles * j

                @pl.when(q < n_rchunk)
                def _():
                    pltpu.make_async_copy(acc.at[pl.ds(q * rch, rch)],
                                          out.at[pl.ds(q * rch, rch)],
                                          wsem).wait()

                return carry

            lax.fori_loop(0, n_rj, wb, 0)
            lax.fori_loop(0, n_rj, wb_drain, 0)

        @pl.when(c == 0)
        def _():
            run(ta, out_a)

        @pl.when(c == 1)
        def _():
            run(tb, out_b)

    return seg_kernel(table_a, table_b, src, dst)


_BN_ROWS = 2000


def _lin1(x, w, b):
    n = x.shape[0]
    d_in, d_out = w.shape
    half = d_out // 2

    def body(x_ref, w_ref, b_ref, oa_ref, ob_ref):
        h = jnp.dot(x_ref[...], w_ref[...], preferred_element_type=F32)
        h = h + b_ref[...]
        oa_ref[...] = h[:, :half]
        ob_ref[...] = h[:, half:]

    return pl.pallas_call(
        body,
        grid=(n // _BN_ROWS,),
        in_specs=[
            pl.BlockSpec((_BN_ROWS, d_in), lambda i: (i, 0)),
            pl.BlockSpec((d_in, d_out), lambda i: (0, 0)),
            pl.BlockSpec((1, d_out), lambda i: (0, 0)),
        ],
        out_specs=[pl.BlockSpec((_BN_ROWS, half), lambda i: (i, 0))] * 2,
        out_shape=[jax.ShapeDtypeStruct((n, half), F32)] * 2,
    )(x, w, b.reshape(1, d_out))


def _gin_mlp(agg_a, agg_b, h_a, h_b, w1, b1, w2, b2):
    n, half = agg_a.shape
    hid = w1.shape[1]
    d_out = w2.shape[1]

    def body(aa, ab, ha, hb, w1_ref, b1_ref, w2_ref, b2_ref,
             t_ref, sum_ref, sq_ref):
        i = pl.program_id(0)
        z = jnp.concatenate([aa[...] + ha[...], ab[...] + hb[...]], axis=1)
        u = jnp.maximum(
            jnp.dot(z, w1_ref[...], preferred_element_type=F32) + b1_ref[...],
            0.0)
        t = jnp.dot(u, w2_ref[...], preferred_element_type=F32) + b2_ref[...]
        t_ref[...] = t

        @pl.when(i == 0)
        def _():
            sum_ref[...] = jnp.zeros_like(sum_ref)
            sq_ref[...] = jnp.zeros_like(sq_ref)

        sum_ref[...] += jnp.sum(t, axis=0, keepdims=True)
        sq_ref[...] += jnp.sum(t * t, axis=0, keepdims=True)

    return pl.pallas_call(
        body,
        grid=(n // _BN_ROWS,),
        in_specs=[
            pl.BlockSpec((_BN_ROWS, half), lambda i: (i, 0)),
            pl.BlockSpec((_BN_ROWS, half), lambda i: (i, 0)),
            pl.BlockSpec((_BN_ROWS, half), lambda i: (i, 0)),
            pl.BlockSpec((_BN_ROWS, half), lambda i: (i, 0)),
            pl.BlockSpec((2 * half, hid), lambda i: (0, 0)),
            pl.BlockSpec((1, hid), lambda i: (0, 0)),
            pl.BlockSpec((hid, d_out), lambda i: (0, 0)),
            pl.BlockSpec((1, d_out), lambda i: (0, 0)),
        ],
        out_specs=[
            pl.BlockSpec((_BN_ROWS, d_out), lambda i: (i, 0)),
            pl.BlockSpec((1, d_out), lambda i: (0, 0)),
            pl.BlockSpec((1, d_out), lambda i: (0, 0)),
        ],
        out_shape=[
            jax.ShapeDtypeStruct((n, d_out), F32),
            jax.ShapeDtypeStruct((1, d_out), F32),
            jax.ShapeDtypeStruct((1, d_out), F32),
        ],
    )(agg_a, agg_b, h_a, h_b, w1, b1.reshape(1, hid), w2, b2.reshape(1, d_out))


def _bn_apply_split(t, t_sum, t_sq, g, b):
    n, d = t.shape
    half = d // 2

    def body(t_ref, sum_ref, sq_ref, g_ref, b_ref, oa_ref, ob_ref):
        m = sum_ref[...] / n
        v = sq_ref[...] / n - m * m
        h = (t_ref[...] - m) * lax.rsqrt(v + 1e-5) * g_ref[...] + b_ref[...]
        oa_ref[...] = h[:, :half]
        ob_ref[...] = h[:, half:]

    return pl.pallas_call(
        body,
        grid=(n // _BN_ROWS,),
        in_specs=[
            pl.BlockSpec((_BN_ROWS, d), lambda i: (i, 0)),
            pl.BlockSpec((1, d), lambda i: (0, 0)),
            pl.BlockSpec((1, d), lambda i: (0, 0)),
            pl.BlockSpec((1, d), lambda i: (0, 0)),
            pl.BlockSpec((1, d), lambda i: (0, 0)),
        ],
        out_specs=[pl.BlockSpec((_BN_ROWS, half), lambda i: (i, 0))] * 2,
        out_shape=[jax.ShapeDtypeStruct((n, half), F32)] * 2,
    )(t, t_sum, t_sq, g.reshape(1, d), b.reshape(1, d))


def _bn_head(t, t_sum, t_sq, g, b, fc1_w, fc1_b, fc2_w, fc2_b):
    n, d = t.shape
    hid = fc1_w.shape[1]
    n_cls = fc2_w.shape[1]

    def body(t_ref, sum_ref, sq_ref, g_ref, b_ref,
             w1_ref, b1_ref, w2_ref, b2_ref, o_ref):
        m = sum_ref[...] / n
        v = sq_ref[...] / n - m * m
        h2 = (t_ref[...] - m) * lax.rsqrt(v + 1e-5) * g_ref[...] + b_ref[...]
        h3 = jnp.maximum(
            jnp.dot(h2, w1_ref[...], preferred_element_type=F32) + b1_ref[...],
            0.0)
        o = (jnp.dot(h3, w2_ref[...], preferred_element_type=F32)
             + b2_ref[...])
        o_ref[...] = o.T

    out_t = pl.pallas_call(
        body,
        out_shape=jax.ShapeDtypeStruct((n_cls, n), F32),
    )(t, t_sum, t_sq, g.reshape(1, d), b.reshape(1, d),
      fc1_w, fc1_b.reshape(1, hid), fc2_w, fc2_b.reshape(1, n_cls))
    return out_t.T


def kernel(x, edge_index, lin1_W, lin1_b, nn1_W1, nn1_b1, nn1_W2, nn1_b2,
           bn1_g, bn1_b, nn2_W1, nn2_b1, nn2_W2, nn2_b2, bn2_g, bn2_b,
           fc1_W, fc1_b, fc2_W, fc2_b):
    src = edge_index[0]
    dst = edge_index[1]

    h_a, h_b = _lin1(x, lin1_W, lin1_b)
    agg_a, agg_b = _sc_segment_sum(h_a, h_b, src, dst)
    t1, s1, q1 = _gin_mlp(agg_a, agg_b, h_a, h_b, nn1_W1, nn1_b1, nn1_W2, nn1_b2)
    h1_a, h1_b = _bn_apply_split(t1, s1, q1, bn1_g, bn1_b)
    a2_a, a2_b = _sc_segment_sum(h1_a, h1_b, src, dst)
    t2, s2, q2 = _gin_mlp(a2_a, a2_b, h1_a, h1_b, nn2_W1, nn2_b1, nn2_W2, nn2_b2)
    return _bn_head(t2, s2, q2, bn2_g, bn2_b, fc1_W, fc1_b, fc2_W, fc2_b)
